# Initial kernel scaffold; baseline (speedup 1.0000x reference)
import jax
import jax.numpy as jnp
from jax.experimental import pallas as pl


def kernel(x, lp_index, lp_values, hp_index, hp_values, W_LP, W_HP, alpha_raw, bias):
    raise NotImplementedError("write your pallas kernel here")


# speedup vs baseline: 3.7665x; 3.7665x over previous
import functools

import jax
import jax.numpy as jnp
from jax import lax
from jax.experimental import pallas as pl
from jax.experimental.pallas import tpu as pltpu
from jax.experimental.pallas import tpu_sc as plsc

_CHUNK = 80
_LANES = 16


def _prep_body(a_ref, x_ref, w_ref, o_ref):
    a = jax.nn.sigmoid(a_ref[0])
    f = pl.program_id(0)
    scale = jnp.where(f == 0, a, 1.0 - a)
    o_ref[...] = scale * jnp.dot(x_ref[...], w_ref[0].T,
                                 preferred_element_type=jnp.float32)


def _finish_body(p_ref, b_ref, o_ref):
    s = p_ref[0] + p_ref[1] + b_ref[...]
    o_ref[...] = jnp.maximum(s, 0.0)


def _make_sc_spmm(N, D, E2):
    NC, NS = 2, 16
    per_tec = E2 // (NC * NS)
    assert per_tec % _CHUNK == 0
    n_chunks = per_tec // _CHUNK
    rows_per_tec = N // NS
    n_zero_chunks = -(-N // _CHUNK)
    zero_rounds = -(-n_zero_chunks // NS)
    groups = _CHUNK // _LANES

    mesh = plsc.VectorSubcoreMesh(core_axis_name="c", subcore_axis_name="s")

    @functools.partial(
        pl.kernel,
        out_type=jax.ShapeDtypeStruct((NC, N, D), jnp.float32),
        mesh=mesh,
        scratch_types=[
            pltpu.VMEM((_CHUNK,), jnp.int32),
            pltpu.VMEM((_CHUNK,), jnp.int32),
            pltpu.VMEM((_CHUNK,), jnp.float32),
            pltpu.VMEM((_CHUNK, D), jnp.float32),
            pltpu.VMEM_SHARED((N, D), jnp.float32),
            pltpu.SemaphoreType.DMA,
        ],
    )
    def sc_spmm(z_hbm, rows_hbm, cols_hbm, vals_hbm, out_hbm,
                col_v, row_v, val_v, rows_v, acc, sem):
        c = lax.axis_index("c")
        s = lax.axis_index("s")
        wid = c * NS + s

        def zero_buf(e, _):
            for j in range(D // _LANES):
                rows_v[e, pl.ds(j * _LANES, _LANES)] = jnp.zeros(
                    (_LANES,), jnp.float32)
            return _

        lax.fori_loop(0, _CHUNK, zero_buf, None)

        def zero_acc(k, _):
            m = s * zero_rounds + k

            @pl.when(m * _CHUNK < N)
            def _():
                sz = jnp.minimum(_CHUNK, N - m * _CHUNK)
                del sz
                pltpu.sync_copy(rows_v, acc.at[pl.ds(m * _CHUNK, _CHUNK)])

            return _

        lax.fori_loop(0, zero_rounds, zero_acc, None)
        plsc.subcore_barrier()

        base = wid * per_tec
        lane_ids = [jnp.full((_LANES, 1), i, dtype=jnp.int32)
                    for i in range(_LANES)]
        _dnums = lax.GatherDimensionNumbers(
            offset_dims=(), collapsed_slice_dims=(0,), start_index_map=(0,))

        def chunk_body(g, _):
            off = base + g * _CHUNK
            pltpu.sync_copy(cols_hbm.at[pl.ds(off, _CHUNK)], col_v)
            pltpu.sync_copy(rows_hbm.at[pl.ds(off, _CHUNK)], row_v)
            pltpu.sync_copy(vals_hbm.at[pl.ds(off, _CHUNK)], val_v)
            pltpu.async_copy(z_hbm.at[col_v], rows_v, sem).wait()

            def scale_group(gg, _c):
                vv = val_v[pl.ds(gg * _LANES, _LANES)]
                e0 = gg * _LANES
                for i in range(_LANES):
                    b = lax.gather(
                        vv, lane_ids[i], _dnums, slice_sizes=(1,),
                        mode=lax.GatherScatterMode.PROMISE_IN_BOUNDS)
                    e = e0 + i
                    for j in range(D // _LANES):
                        sl = pl.ds(j * _LANES, _LANES)
                        rows_v[e, sl] = rows_v[e, sl] * b
                return _c

            lax.fori_loop(0, groups, scale_group, None)
            pltpu.sync_copy(rows_v, acc.at[row_v], add=True)
            return _

        lax.fori_loop(0, n_chunks, chunk_body, None)
        plsc.subcore_barrier()

        def write_out(k, _):
            m = s * zero_rounds + k

            @pl.when(m * _CHUNK < N)
            def _():
                pltpu.sync_copy(acc.at[pl.ds(m * _CHUNK, _CHUNK)],
                                out_hbm.at[c, pl.ds(m * _CHUNK, _CHUNK)])

            return _

        lax.fori_loop(0, zero_rounds, write_out, None)

    return sc_spmm


def kernel(x, lp_index, lp_values, hp_index, hp_values, W_LP, W_HP,
           alpha_raw, bias):
    N, D = x.shape
    E = lp_values.shape[0]
    BN = 2000
    NB = N // BN

    Ws = jnp.stack([W_LP, W_HP])

    Z = pl.pallas_call(
        _prep_body,
        grid=(2, NB),
        in_specs=[
            pl.BlockSpec(memory_space=pltpu.SMEM),
            pl.BlockSpec((BN, D), lambda f, b: (b, 0)),
            pl.BlockSpec((1, D, D), lambda f, b: (f, 0, 0)),
        ],
        out_specs=pl.BlockSpec((BN, D), lambda f, b: (f * NB + b, 0)),
        out_shape=jax.ShapeDtypeStruct((2 * N, D), jnp.float32),
    )(alpha_raw, x, Ws)

    rows = jnp.concatenate([lp_index[0], hp_index[0]])
    cols = jnp.concatenate([lp_index[1], hp_index[1] + N])
    vals = jnp.concatenate([lp_values, hp_values])

    parts = _make_sc_spmm(N, D, 2 * E)(Z, rows, cols, vals)

    out = pl.pallas_call(
        _finish_body,
        grid=(NB,),
        in_specs=[
            pl.BlockSpec((2, BN, D), lambda b: (0, b, 0)),
            pl.BlockSpec((1, D), lambda b: (0, 0)),
        ],
        out_specs=pl.BlockSpec((BN, D), lambda b: (b, 0)),
        out_shape=jax.ShapeDtypeStruct((N, D), jnp.float32),
    )(parts, bias.reshape(1, D))

    return out


# speedup vs baseline: 8.7255x; 2.3166x over previous
import functools

import jax
import jax.numpy as jnp
from jax import lax
from jax.experimental import pallas as pl
from jax.experimental.pallas import tpu as pltpu
from jax.experimental.pallas import tpu_sc as plsc

_CHUNK = 80
_BATCH = 50
_LANES = 16


def _prep_body(a_ref, x_ref, w_ref, o_ref):
    a = jax.nn.sigmoid(a_ref[0])
    f = pl.program_id(0)
    scale = jnp.where(f == 0, a, 1.0 - a)
    o_ref[...] = scale * jnp.dot(x_ref[...], w_ref[0].T,
                                 preferred_element_type=jnp.float32)


def _finish_body(p_ref, b_ref, o_ref):
    s = p_ref[0] + p_ref[1] + b_ref[...]
    o_ref[...] = jnp.maximum(s, 0.0)


def _make_sc_spmm(N, D, E2):
    NC, NS = 2, 16
    per_tec = E2 // (NC * NS)
    assert per_tec % _CHUNK == 0
    n_chunks = per_tec // _CHUNK
    assert n_chunks % _BATCH == 0
    rows_per_tec = N // NS
    n_zero_chunks = -(-N // _CHUNK)
    zero_rounds = -(-n_zero_chunks // NS)
    groups = _CHUNK // _LANES

    mesh = plsc.VectorSubcoreMesh(core_axis_name="c", subcore_axis_name="s")

    @functools.partial(
        pl.kernel,
        out_type=jax.ShapeDtypeStruct((NC, N, D), jnp.float32),
        mesh=mesh,
        scratch_types=[
            pltpu.VMEM((_BATCH, _CHUNK), jnp.int32),
            pltpu.VMEM((_BATCH, _CHUNK), jnp.int32),
            pltpu.VMEM((_BATCH, _CHUNK), jnp.float32),
            pltpu.VMEM((_CHUNK, D), jnp.float32),
            pltpu.VMEM((_CHUNK, D), jnp.float32),
            pltpu.VMEM_SHARED((N, D), jnp.float32),
            pltpu.SemaphoreType.DMA,
            pltpu.SemaphoreType.DMA,
        ],
    )
    def sc_spmm(z_hbm, rows_hbm, cols_hbm, vals_hbm, out_hbm,
                col_v, row_v, val_v, rows_v0, rows_v1, acc, sem0, sem1):
        c = lax.axis_index("c")
        s = lax.axis_index("s")
        wid = c * NS + s
        bufs = (rows_v0, rows_v1)
        sems = (sem0, sem1)

        def zero_buf(e, _):
            for j in range(D // _LANES):
                rows_v0[e, pl.ds(j * _LANES, _LANES)] = jnp.zeros(
                    (_LANES,), jnp.float32)
            return _

        lax.fori_loop(0, _CHUNK, zero_buf, None)

        def zero_acc(k, _):
            m = s * zero_rounds + k

            @pl.when(m * _CHUNK < N)
            def _():
                pltpu.sync_copy(rows_v0, acc.at[pl.ds(m * _CHUNK, _CHUNK)])

            return _

        lax.fori_loop(0, zero_rounds, zero_acc, None)
        plsc.subcore_barrier()

        lane_ids = [jnp.full((_LANES, 1), i, dtype=jnp.int32)
                    for i in range(_LANES)]
        _dnums = lax.GatherDimensionNumbers(
            offset_dims=(), collapsed_slice_dims=(0,), start_index_map=(0,))

        def scale(buf, g):
            def scale_group(gg, _c):
                vv = val_v[g, pl.ds(gg * _LANES, _LANES)]
                for i in range(_LANES):
                    b = lax.gather(
                        vv, lane_ids[i], _dnums, slice_sizes=(1,),
                        mode=lax.GatherScatterMode.PROMISE_IN_BOUNDS)
                    e = gg * _LANES + i
                    for j in range(D // _LANES):
                        sl = pl.ds(j * _LANES, _LANES)
                        buf[e, sl] = buf[e, sl] * b
                return _c

            lax.fori_loop(0, groups, scale_group, None)

        def batch_body(bt, _):
            pltpu.sync_copy(cols_hbm.at[wid, bt], col_v)
            pltpu.sync_copy(rows_hbm.at[wid, bt], row_v)
            pltpu.sync_copy(vals_hbm.at[wid, bt], val_v)
            pltpu.async_copy(z_hbm.at[col_v.at[0]], rows_v0, sem0)

            def pair_body(p, _c):
                for b in range(2):
                    g = 2 * p + b
                    ob = 1 - b
                    gnext = jnp.minimum(g + 1, _BATCH - 1)
                    pltpu.async_copy(z_hbm.at[col_v.at[gnext]], bufs[ob],
                                     sems[ob])
                    pltpu.make_async_copy(z_hbm.at[col_v.at[g]], bufs[b],
                                          sems[b]).wait()
                    scale(bufs[b], g)
                    pltpu.sync_copy(bufs[b], acc.at[row_v.at[g]], add=True)
                return _c

            lax.fori_loop(0, _BATCH // 2, pair_body, None)
            pltpu.make_async_copy(z_hbm.at[col_v.at[0]], rows_v0, sem0).wait()
            return _

        lax.fori_loop(0, n_chunks // _BATCH, batch_body, None)
        plsc.subcore_barrier()

        def write_out(k, _):
            m = s * zero_rounds + k

            @pl.when(m * _CHUNK < N)
            def _():
                pltpu.sync_copy(acc.at[pl.ds(m * _CHUNK, _CHUNK)],
                                out_hbm.at[c, pl.ds(m * _CHUNK, _CHUNK)])

            return _

        lax.fori_loop(0, zero_rounds, write_out, None)

    return sc_spmm


def kernel(x, lp_index, lp_values, hp_index, hp_values, W_LP, W_HP,
           alpha_raw, bias):
    N, D = x.shape
    E = lp_values.shape[0]
    BN = 2000
    NB = N // BN

    Ws = jnp.stack([W_LP, W_HP])

    Z = pl.pallas_call(
        _prep_body,
        grid=(2, NB),
        in_specs=[
            pl.BlockSpec(memory_space=pltpu.SMEM),
            pl.BlockSpec((BN, D), lambda f, b: (b, 0)),
            pl.BlockSpec((1, D, D), lambda f, b: (f, 0, 0)),
        ],
        out_specs=pl.BlockSpec((BN, D), lambda f, b: (f * NB + b, 0)),
        out_shape=jax.ShapeDtypeStruct((2 * N, D), jnp.float32),
    )(alpha_raw, x, Ws)

    NW = 32
    nbt = (2 * E) // (NW * _CHUNK * _BATCH)
    shp = (NW, nbt, _BATCH, _CHUNK)
    rows = jnp.concatenate([lp_index[0], hp_index[0]]).reshape(shp)
    cols = jnp.concatenate([lp_index[1], hp_index[1] + N]).reshape(shp)
    vals = jnp.concatenate([lp_values, hp_values]).reshape(shp)

    parts = _make_sc_spmm(N, D, 2 * E)(Z, rows, cols, vals)

    out = pl.pallas_call(
        _finish_body,
        grid=(NB,),
        in_specs=[
            pl.BlockSpec((2, BN, D), lambda b: (0, b, 0)),
            pl.BlockSpec((1, D), lambda b: (0, 0)),
        ],
        out_specs=pl.BlockSpec((BN, D), lambda b: (b, 0)),
        out_shape=jax.ShapeDtypeStruct((N, D), jnp.float32),
    )(parts, bias.reshape(1, D))

    return out
